# baseline (speedup 1.0000x reference)
import functools

import jax
import jax.numpy as jnp
from jax import lax
from jax.experimental import pallas as pl
from jax.experimental.pallas import tpu as pltpu
from jax.experimental.pallas import tpu_sc as plsc

B = 4096
L = 20
DIM = 64
NC = 2
NS = 16
NW = NC * NS
RPW = B // NW
NV = DIM // 16
NBUF = 8
NRV = RPW // 16


def _sc_pool(s_t, o_t, p, W_words, W_rel):
    mesh = plsc.VectorSubcoreMesh(core_axis_name="c", subcore_axis_name="s")

    @functools.partial(
        pl.kernel,
        out_type=[jax.ShapeDtypeStruct((B, DIM), jnp.float32)] * 3,
        mesh=mesh,
        compiler_params=pltpu.CompilerParams(use_tc_tiling_on_sc=False,
                                             needs_layout_passes=False),
        scratch_types=[
            pltpu.VMEM((L, RPW), jnp.int32),
            pltpu.VMEM((RPW, DIM), jnp.float32),
            pltpu.VMEM((RPW, DIM), jnp.float32),
            pltpu.VMEM((RPW, DIM), jnp.float32),
            pltpu.VMEM((RPW, DIM), jnp.float32),
            pltpu.VMEM((RPW, DIM), jnp.float32),
            pltpu.VMEM((RPW, DIM), jnp.float32),
            pltpu.VMEM((RPW, DIM), jnp.float32),
            pltpu.VMEM((RPW, DIM), jnp.float32),
            pltpu.VMEM((RPW, DIM), jnp.float32),
            pltpu.VMEM((RPW,), jnp.float32),
            pltpu.VMEM((RPW,), jnp.int32),
            pltpu.VMEM((RPW, DIM), jnp.float32),
            pltpu.SemaphoreType.DMA,
            pltpu.SemaphoreType.DMA,
            pltpu.SemaphoreType.DMA,
            pltpu.SemaphoreType.DMA,
            pltpu.SemaphoreType.DMA,
            pltpu.SemaphoreType.DMA,
            pltpu.SemaphoreType.DMA,
            pltpu.SemaphoreType.DMA,
            pltpu.SemaphoreType.DMA,
        ],
    )
    def sc_kernel(s_hbm, o_hbm, p_hbm, ww_hbm, wr_hbm,
                  s_out, o_out, p_out,
                  idx_v, gbuf0, gbuf1, gbuf2, gbuf3, gbuf4, gbuf5, gbuf6,
                  gbuf7, out_v, freq_v, pidx_v, prow_v,
                  sem0, sem1, sem2, sem3, sem4, sem5, sem6, sem7, psem):
        wid = lax.axis_index("s") * NC + lax.axis_index("c")
        base = wid * RPW
        bufs = (gbuf0, gbuf1, gbuf2, gbuf3, gbuf4, gbuf5, gbuf6, gbuf7)
        sems = (sem0, sem1, sem2, sem3, sem4, sem5, sem6, sem7)

        pltpu.sync_copy(p_hbm.at[pl.ds(base, RPW)], pidx_v)
        pltpu.make_async_copy(wr_hbm.at[pidx_v], prow_v, psem).start()

        def run_side(side_hbm, side_out):
            pltpu.sync_copy(side_hbm.at[:, pl.ds(base, RPW)], idx_v)

            def start(l, b):
                pltpu.make_async_copy(ww_hbm.at[idx_v.at[l]], bufs[b],
                                      sems[b]).start()

            def wait(b):
                pltpu.make_async_copy(ww_hbm.at[idx_v.at[0]], bufs[b],
                                      sems[b]).wait()

            for b in range(6):
                start(b, b)

            for rv in range(NRV):
                sl = pl.ds(rv * 16, 16)
                cnt = jnp.zeros((16,), jnp.int32)
                for li in range(L):
                    cnt = cnt + jnp.minimum(idx_v[li, sl], 1)
                freq_v[sl] = cnt.astype(jnp.float32)


            def zero_body(i, carry):
                for u in range(4):
                    r = i * 4 + u
                    for d in range(NV):
                        out_v[r, pl.ds(d * 16, 16)] = jnp.zeros(
                            (16,), jnp.float32)
                return carry

            lax.fori_loop(0, RPW // 4, zero_body, 0)

            def reduce_slot(gb):
                def row_body(i, c2, gb=gb):
                    for u in range(4):
                        r = i * 4 + u
                        for d in range(NV):
                            plsc.addupdate(
                                out_v.at[r, pl.ds(d * 16, 16)],
                                gb[r, pl.ds(d * 16, 16)])
                    return c2

                lax.fori_loop(0, RPW // 4, row_body, 0)

            def acc_body(g, carry):
                for b in range(NBUF):
                    l = g * NBUF + b
                    wait(b)

                    @pl.when(l + 6 < L)
                    def _():
                        start(l + 6, (b + 6) % NBUF)

                    reduce_slot(bufs[b])
                return carry

            lax.fori_loop(0, (L // NBUF) * NBUF // NBUF, acc_body, 0)
            for l in range((L // NBUF) * NBUF, L):
                wait(l % NBUF)
                reduce_slot(bufs[l % NBUF])

            def scale_body(i, carry):
                for u in range(2):
                    r = i * 2 + u
                    fspl = plsc.load_gather(freq_v,
                                            [jnp.full((16,), r, jnp.int32)])
                    for d in range(NV):
                        sl = pl.ds(d * 16, 16)
                        out_v[r, sl] = out_v[r, sl] * fspl
                return carry

            lax.fori_loop(0, RPW // 2, scale_body, 0)
            pltpu.sync_copy(out_v, side_out.at[pl.ds(base, RPW)])

        run_side(s_hbm, s_out)
        run_side(o_hbm, o_out)

        pltpu.make_async_copy(wr_hbm.at[pidx_v], prow_v, psem).wait()
        pltpu.sync_copy(prow_v, p_out.at[pl.ds(base, RPW)])

    return sc_kernel(s_t, o_t, p, W_words, W_rel)


def _tc_dense(se_f, oe_f, pe_f, W2blk, b2, E2):
    def tc_kernel(se_ref, oe_ref, pe_ref, w_ref, b_ref, e_ref, out_ref):
        w2 = w_ref[...]
        b2 = b_ref[...]
        st = jnp.maximum(
            jnp.dot(se_ref[...], w2, preferred_element_type=jnp.float32)
            + b2, 0.0)
        ot = jnp.maximum(
            jnp.dot(oe_ref[...], w2, preferred_element_type=jnp.float32)
            + b2, 0.0)
        prod = st * pe_ref[...] * ot
        out_ref[...] = jnp.dot(prod, e_ref[...],
                               preferred_element_type=jnp.float32)

    return pl.pallas_call(
        tc_kernel,
        out_shape=jax.ShapeDtypeStruct((B // 2, 2), jnp.float32),
    )(se_f, oe_f, pe_f, W2blk, b2, E2)


def kernel(s, o, p, W_words, W_rel, W_lin, b_lin):
    s_t = s.T.astype(jnp.int32)
    o_t = o.T.astype(jnp.int32)
    p_i = p.astype(jnp.int32)
    s_emb, o_emb, p_emb = _sc_pool(s_t, o_t, p_i, W_words, W_rel)
    se_f = s_emb.reshape(B // 2, 2 * DIM)
    oe_f = o_emb.reshape(B // 2, 2 * DIM)
    pe_f = p_emb.reshape(B // 2, 2 * DIM)
    wt = W_lin.T
    W2blk = jnp.zeros((2 * DIM, 2 * DIM), jnp.float32)
    W2blk = W2blk.at[:DIM, :DIM].set(wt).at[DIM:, DIM:].set(wt)
    b2 = jnp.concatenate([b_lin, b_lin]).reshape(1, 2 * DIM)
    half = jnp.arange(2 * DIM) >= DIM
    E2 = jnp.stack([(~half).astype(jnp.float32),
                    half.astype(jnp.float32)], axis=1)
    pred_f = _tc_dense(se_f, oe_f, pe_f, W2blk, b2, E2)
    return pred_f.reshape(B)


# speedup vs baseline: 1.0348x; 1.0348x over previous
import functools

import jax
import jax.numpy as jnp
from jax import lax
from jax.experimental import pallas as pl
from jax.experimental.pallas import tpu as pltpu
from jax.experimental.pallas import tpu_sc as plsc

B = 4096
L = 20
DIM = 64
NC = 2
NS = 16
NW = NC * NS
RPW = B // NW
NV = DIM // 16
NBUF = 4
NRV = RPW // 16


def _sc_pool(s_t, o_t, p, W_words, W_rel):
    mesh = plsc.VectorSubcoreMesh(core_axis_name="c", subcore_axis_name="s")

    @functools.partial(
        pl.kernel,
        out_type=[jax.ShapeDtypeStruct((B, DIM), jnp.float32)] * 3,
        mesh=mesh,
        compiler_params=pltpu.CompilerParams(use_tc_tiling_on_sc=False,
                                             needs_layout_passes=False),
        scratch_types=[
            pltpu.VMEM((L, RPW), jnp.int32),
            pltpu.VMEM((RPW, DIM), jnp.float32),
            pltpu.VMEM((RPW, DIM), jnp.float32),
            pltpu.VMEM((RPW, DIM), jnp.float32),
            pltpu.VMEM((RPW, DIM), jnp.float32),
            pltpu.VMEM((RPW, DIM), jnp.float32),
            pltpu.VMEM((RPW,), jnp.float32),
            pltpu.VMEM((RPW,), jnp.int32),
            pltpu.VMEM((RPW, DIM), jnp.float32),
            pltpu.SemaphoreType.DMA,
            pltpu.SemaphoreType.DMA,
            pltpu.SemaphoreType.DMA,
            pltpu.SemaphoreType.DMA,
            pltpu.SemaphoreType.DMA,
        ],
    )
    def sc_kernel(s_hbm, o_hbm, p_hbm, ww_hbm, wr_hbm,
                  s_out, o_out, p_out,
                  idx_v, gbuf0, gbuf1, gbuf2, gbuf3, out_v, freq_v,
                  pidx_v, prow_v, sem0, sem1, sem2, sem3, psem):
        wid = lax.axis_index("s") * NC + lax.axis_index("c")
        base = wid * RPW
        bufs = (gbuf0, gbuf1, gbuf2, gbuf3)
        sems = (sem0, sem1, sem2, sem3)

        pltpu.sync_copy(p_hbm.at[pl.ds(base, RPW)], pidx_v)
        pltpu.make_async_copy(wr_hbm.at[pidx_v], prow_v, psem).start()

        def run_side(side_hbm, side_out):
            pltpu.sync_copy(side_hbm.at[:, pl.ds(base, RPW)], idx_v)

            def start(l, b):
                pltpu.make_async_copy(ww_hbm.at[idx_v.at[l]], bufs[b],
                                      sems[b]).start()

            def wait(b):
                pltpu.make_async_copy(ww_hbm.at[idx_v.at[0]], bufs[b],
                                      sems[b]).wait()

            for b in range(NBUF - 1):
                start(b, b)

            for rv in range(NRV):
                sl = pl.ds(rv * 16, 16)
                cnt = jnp.zeros((16,), jnp.int32)
                for li in range(L):
                    cnt = cnt + jnp.minimum(idx_v[li, sl], 1)
                freq_v[sl] = cnt.astype(jnp.float32)


            def zero_body(i, carry):
                for u in range(4):
                    r = i * 4 + u
                    for d in range(NV):
                        out_v[r, pl.ds(d * 16, 16)] = jnp.zeros(
                            (16,), jnp.float32)
                return carry

            lax.fori_loop(0, RPW // 4, zero_body, 0)

            def acc_body(g, carry):
                for b in range(NBUF):
                    l = g * NBUF + b
                    wait(b)
                    gb = bufs[b]

                    nxt = l + NBUF - 1

                    @pl.when(nxt < L)
                    def _():
                        start(nxt, (b + NBUF - 1) % NBUF)

                    def row_body(i, c2, gb=gb):
                        for u in range(4):
                            r = i * 4 + u
                            for d in range(NV):
                                plsc.addupdate(
                                    out_v.at[r, pl.ds(d * 16, 16)],
                                    gb[r, pl.ds(d * 16, 16)])
                        return c2

                    lax.fori_loop(0, RPW // 4, row_body, 0)
                return carry

            lax.fori_loop(0, L // NBUF, acc_body, 0)

            def scale_body(i, carry):
                for u in range(2):
                    r = i * 2 + u
                    fspl = plsc.load_gather(freq_v,
                                            [jnp.full((16,), r, jnp.int32)])
                    for d in range(NV):
                        sl = pl.ds(d * 16, 16)
                        out_v[r, sl] = out_v[r, sl] * fspl
                return carry

            lax.fori_loop(0, RPW // 2, scale_body, 0)
            pltpu.sync_copy(out_v, side_out.at[pl.ds(base, RPW)])

        run_side(s_hbm, s_out)
        run_side(o_hbm, o_out)

        pltpu.make_async_copy(wr_hbm.at[pidx_v], prow_v, psem).wait()
        pltpu.sync_copy(prow_v, p_out.at[pl.ds(base, RPW)])

    return sc_kernel(s_t, o_t, p, W_words, W_rel)


def _tc_dense(se_f, oe_f, pe_f, W2blk, b2, E2):
    def tc_kernel(se_ref, oe_ref, pe_ref, w_ref, b_ref, e_ref, out_ref):
        w2 = w_ref[...]
        b2 = b_ref[...]
        st = jnp.maximum(
            jnp.dot(se_ref[...], w2, preferred_element_type=jnp.float32)
            + b2, 0.0)
        ot = jnp.maximum(
            jnp.dot(oe_ref[...], w2, preferred_element_type=jnp.float32)
            + b2, 0.0)
        prod = st * pe_ref[...] * ot
        out_ref[...] = jnp.dot(prod, e_ref[...],
                               preferred_element_type=jnp.float32)

    return pl.pallas_call(
        tc_kernel,
        out_shape=jax.ShapeDtypeStruct((B // 2, 2), jnp.float32),
    )(se_f, oe_f, pe_f, W2blk, b2, E2)


def kernel(s, o, p, W_words, W_rel, W_lin, b_lin):
    s_t = s.T.astype(jnp.int32)
    o_t = o.T.astype(jnp.int32)
    p_i = p.astype(jnp.int32)
    s_emb, o_emb, p_emb = _sc_pool(s_t, o_t, p_i, W_words, W_rel)
    se_f = s_emb.reshape(B // 2, 2 * DIM)
    oe_f = o_emb.reshape(B // 2, 2 * DIM)
    pe_f = p_emb.reshape(B // 2, 2 * DIM)
    wt = W_lin.T
    W2blk = jnp.zeros((2 * DIM, 2 * DIM), jnp.float32)
    W2blk = W2blk.at[:DIM, :DIM].set(wt).at[DIM:, DIM:].set(wt)
    b2 = jnp.concatenate([b_lin, b_lin]).reshape(1, 2 * DIM)
    half = jnp.arange(2 * DIM) >= DIM
    E2 = jnp.stack([(~half).astype(jnp.float32),
                    half.astype(jnp.float32)], axis=1)
    pred_f = _tc_dense(se_f, oe_f, pe_f, W2blk, b2, E2)
    return pred_f.reshape(B)
